# Initial kernel scaffold; baseline (speedup 1.0000x reference)
import jax
import jax.numpy as jnp
from jax.experimental import pallas as pl


def kernel(x):
    raise NotImplementedError("write your pallas kernel here")


# speedup vs baseline: 6.3718x; 6.3718x over previous
import functools

import jax
import jax.numpy as jnp
from jax import lax
from jax.experimental import pallas as pl
from jax.experimental.pallas import tpu as pltpu
from jax.experimental.pallas import tpu_sc as plsc

ROWS = 128
N = 32768
LANES = 16
NCHUNK = N // LANES
NUM_CORES = 2
NUM_SUBCORES = 16
NUM_WORKERS = NUM_CORES * NUM_SUBCORES
ROWS_PER_W = ROWS // NUM_WORKERS

_mesh = plsc.VectorSubcoreMesh(
    core_axis_name="c", subcore_axis_name="s",
    num_cores=NUM_CORES, num_subcores=NUM_SUBCORES)


def _chunk(ref, j):
    return ref[pl.ds(pl.multiple_of(j * LANES, LANES), LANES)]


def _sparsemax_body(x_hbm, out_hbm, row_v, cand_v):
    wid = lax.axis_index("s") * NUM_CORES + lax.axis_index("c")

    def do_row(i, carry):
        r = wid * ROWS_PER_W + i
        pltpu.sync_copy(x_hbm.at[r], row_v)

        def amax(j, acc):
            return jnp.maximum(acc, _chunk(row_v, j))
        acc = lax.fori_loop(0, NCHUNK, amax,
                            jnp.full((LANES,), -jnp.inf, jnp.float32))
        m = jnp.max(acc)
        thr = m - 1.0

        def compact(j, off_vec):
            v = _chunk(row_v, j)
            msk = v > thr

            def have(ov):
                pos = plsc.cumsum(msk.astype(jnp.int32)) - 1 + ov
                plsc.store_scatter(cand_v, [pos], v, mask=msk)
                return ov + plsc.all_reduce_population_count(msk)

            return lax.cond(jnp.any(msk), have, lambda ov: ov, off_vec)

        off_vec = lax.fori_loop(0, NCHUNK, compact,
                                jnp.zeros((LANES,), jnp.int32))
        pad_idx = off_vec + lax.iota(jnp.int32, LANES)
        plsc.store_scatter(cand_v, [pad_idx],
                           jnp.full((LANES,), thr, jnp.float32))
        k_cand = jnp.max(off_vec)
        nch = lax.shift_right_logical(k_cand + (LANES - 1), 4)

        def fsum(t):
            def body(j, s):
                return s + jnp.maximum(_chunk(cand_v, j) - t, 0.0)
            sv = lax.fori_loop(0, nch, body, jnp.zeros((LANES,), jnp.float32))
            return jnp.sum(sv)

        def bis(it, lohi):
            lo, hi = lohi
            mid = 0.5 * (lo + hi)
            gt = fsum(mid) > 1.0
            return (jnp.where(gt, mid, lo), jnp.where(gt, hi, mid))

        lo, _ = lax.fori_loop(0, 30, bis, (thr, m))

        def sc_body(j, carry2):
            s, c = carry2
            v = _chunk(cand_v, j)
            msk = v > lo
            return (s + jnp.where(msk, v, 0.0), c + msk.astype(jnp.int32))
        sv, cv = lax.fori_loop(
            0, nch, sc_body,
            (jnp.zeros((LANES,), jnp.float32), jnp.zeros((LANES,), jnp.int32)))
        s_v = jnp.full((LANES,), jnp.sum(sv) - 1.0, jnp.float32)
        c_v = jnp.full((LANES,), jnp.sum(cv), jnp.int32).astype(jnp.float32)
        tau_v = s_v / c_v

        def outp(j, c2):
            jslice = pl.ds(pl.multiple_of(j * LANES, LANES), LANES)
            row_v[jslice] = jnp.maximum(row_v[jslice] - tau_v, 0.0)
            return c2
        lax.fori_loop(0, NCHUNK, outp, 0)
        pltpu.sync_copy(row_v, out_hbm.at[r])
        return carry

    lax.fori_loop(0, ROWS_PER_W, do_row, 0)


_sparsemax = functools.partial(
    pl.kernel,
    out_type=jax.ShapeDtypeStruct((ROWS, N), jnp.float32),
    mesh=_mesh,
    scratch_types=[
        pltpu.VMEM((N,), jnp.float32),
        pltpu.VMEM((N + LANES,), jnp.float32),
    ],
    compiler_params=pltpu.CompilerParams(needs_layout_passes=False),
)(_sparsemax_body)


@jax.jit
def kernel(x):
    return _sparsemax(x)


# speedup vs baseline: 30.2911x; 4.7539x over previous
import functools

import jax
import jax.numpy as jnp
from jax import lax
from jax.experimental import pallas as pl
from jax.experimental.pallas import tpu as pltpu
from jax.experimental.pallas import tpu_sc as plsc

ROWS = 128
N = 32768
LANES = 16
NCHUNK = N // LANES
NUM_CORES = 2
NUM_SUBCORES = 16
NUM_WORKERS = NUM_CORES * NUM_SUBCORES
ROWS_PER_W = ROWS // NUM_WORKERS

_mesh = plsc.VectorSubcoreMesh(
    core_axis_name="c", subcore_axis_name="s",
    num_cores=NUM_CORES, num_subcores=NUM_SUBCORES)


def _chunk(ref, j):
    return ref[pl.ds(pl.multiple_of(j * LANES, LANES), LANES)]


def _sparsemax_body(x_hbm, out_hbm, row_v, cand_v):
    wid = lax.axis_index("s") * NUM_CORES + lax.axis_index("c")

    def do_row(i, carry):
        r = wid * ROWS_PER_W + i
        pltpu.sync_copy(x_hbm.at[r], row_v)

        @plsc.parallel_loop(0, N, step=LANES, unroll=8,
                            carry=jnp.full((LANES,), -jnp.inf, jnp.float32))
        def acc(i, a):
            return jnp.maximum(a, row_v[pl.ds(pl.multiple_of(i, LANES), LANES)])
        m = jnp.max(acc)
        thr = m - 1.0

        @plsc.parallel_loop(0, N, step=LANES, unroll=8,
                            carry=jnp.zeros((LANES,), jnp.int32))
        def off_vec(i, ov):
            v = row_v[pl.ds(pl.multiple_of(i, LANES), LANES)]
            msk = v > thr

            def have(o):
                pos = plsc.cumsum(msk.astype(jnp.int32)) - 1 + o
                plsc.store_scatter(cand_v, [pos], v, mask=msk)
                return o + plsc.all_reduce_population_count(msk)

            return lax.cond(jnp.any(msk), have, lambda o: o, ov)
        pad_idx = off_vec + lax.iota(jnp.int32, LANES)
        plsc.store_scatter(cand_v, [pad_idx],
                           jnp.full((LANES,), thr, jnp.float32))
        k_cand = jnp.max(off_vec)
        nch = lax.shift_right_logical(k_cand + (LANES - 1), 4)

        def fsum(t):
            def body(j, s):
                return s + jnp.maximum(_chunk(cand_v, j) - t, 0.0)
            sv = lax.fori_loop(0, nch, body, jnp.zeros((LANES,), jnp.float32))
            return jnp.sum(sv)

        def bis(it, lohi):
            lo, hi = lohi
            mid = 0.5 * (lo + hi)
            gt = fsum(mid) > 1.0
            return (jnp.where(gt, mid, lo), jnp.where(gt, hi, mid))

        lo, _ = lax.fori_loop(0, 30, bis, (thr, m))

        def sc_body(j, carry2):
            s, c = carry2
            v = _chunk(cand_v, j)
            msk = v > lo
            return (s + jnp.where(msk, v, 0.0), c + msk.astype(jnp.int32))
        sv, cv = lax.fori_loop(
            0, nch, sc_body,
            (jnp.zeros((LANES,), jnp.float32), jnp.zeros((LANES,), jnp.int32)))
        s_v = jnp.full((LANES,), jnp.sum(sv) - 1.0, jnp.float32)
        c_v = jnp.full((LANES,), jnp.sum(cv), jnp.int32).astype(jnp.float32)
        tau_v = s_v / c_v

        @plsc.parallel_loop(0, N, step=LANES, unroll=8)
        def _(i):
            jslice = pl.ds(pl.multiple_of(i, LANES), LANES)
            row_v[jslice] = jnp.maximum(row_v[jslice] - tau_v, 0.0)
        pltpu.sync_copy(row_v, out_hbm.at[r])
        return carry

    lax.fori_loop(0, ROWS_PER_W, do_row, 0)


_sparsemax = functools.partial(
    pl.kernel,
    out_type=jax.ShapeDtypeStruct((ROWS, N), jnp.float32),
    mesh=_mesh,
    scratch_types=[
        pltpu.VMEM((N,), jnp.float32),
        pltpu.VMEM((N + LANES,), jnp.float32),
    ],
    compiler_params=pltpu.CompilerParams(needs_layout_passes=False),
)(_sparsemax_body)


@jax.jit
def kernel(x):
    return _sparsemax(x)
